# baseline (speedup 1.0000x reference)
import functools

import jax
import jax.numpy as jnp
import numpy as np
from jax.experimental import pallas as pl
from jax.experimental.pallas import tpu as pltpu

_CONV_SPECS = [(1024, 256, 256, 0), (512, 128, 128, 0), (256, 64, 64, 0),
               (128, 32, 32, 0), (64, 16, 16, 0), (32, 8, 8, 0),
               (16, 4, 4, 0), (8, 3, 2, 1)]
_NC = 1000
_EPS = 1e-5
_GRAY = np.array([0.299, 0.587, 0.114], np.float32)
_RES = [1, 2, 4, 8, 16, 32, 64, 128]


def _mm_bn_silu_kernel(nk, p_ref, w_ref, s_ref, t_ref, o_ref, acc_ref):
    k = pl.program_id(2)

    @pl.when(k == 0)
    def _():
        acc_ref[...] = jnp.zeros_like(acc_ref)

    acc_ref[...] += jax.lax.dot_general(
        p_ref[...], w_ref[...], (((1,), (1,)), ((), ())),
        preferred_element_type=jnp.float32)

    @pl.when(k == nk - 1)
    def _():
        y = acc_ref[...] * s_ref[...] + t_ref[...]
        o_ref[...] = y * jax.nn.sigmoid(y)


def _mm_bn_silu(patches, wt, scale, shift, mt, nt, kt):
    M, K = patches.shape
    N = wt.shape[0]
    nm, nn, nk = M // mt, N // nt, K // kt
    return pl.pallas_call(
        functools.partial(_mm_bn_silu_kernel, nk),
        grid=(nm, nn, nk),
        in_specs=[
            pl.BlockSpec((mt, kt), lambda m, n, k: (m, k)),
            pl.BlockSpec((nt, kt), lambda m, n, k: (n, k)),
            pl.BlockSpec((1, nt), lambda m, n, k: (0, n)),
            pl.BlockSpec((1, nt), lambda m, n, k: (0, n)),
        ],
        out_specs=pl.BlockSpec((mt, nt), lambda m, n, k: (m, n)),
        out_shape=jax.ShapeDtypeStruct((M, N), jnp.float32),
        scratch_shapes=[pltpu.VMEM((mt, nt), jnp.float32)],
        compiler_params=pltpu.CompilerParams(
            dimension_semantics=("parallel", "parallel", "arbitrary")),
    )(patches, wt, scale[None], shift[None])


_CONV_TILES = [(2, 512, 2048), (8, 256, 4096), (16, 256, 4096),
               (64, 128, 1024), (256, 64, 256), (1024, 32, 64),
               (4096, 16, 16), (16384, 8, 16)]


def _mlp_stats_kernel(*refs):
    feats = refs[0:8]
    w1s = refs[8:16]
    b1, lng, lnb, w2, b2 = refs[16:21]
    pred_ref, norm_ref = refs[21:23]

    rowbase = None
    for s in range(5):
        res = _RES[s]
        f = 128 // res
        cells = feats[s][0].reshape(res, -1)
        g = jnp.dot(cells, w1s[s][...], preferred_element_type=jnp.float32)
        g = jnp.broadcast_to(g[:, None, :], (res, f, 1032)).reshape(128, 1032)
        rowbase = g if rowbase is None else rowbase + g
    acc = jnp.broadcast_to(rowbase[None], (8, 128, 1032)).reshape(1024, 1032)
    for s in range(5, 8):
        res = _RES[s]
        f = 128 // res
        cr = 8 // f
        cells = feats[s][0].reshape(cr * res, -1)
        g = jnp.dot(cells, w1s[s][...], preferred_element_type=jnp.float32)
        if f == 1:
            acc += g
        else:
            g = g.reshape(cr, res, 1032)
            g = jnp.broadcast_to(g[:, None, :, None, :], (cr, f, res, f, 1032))
            acc += g.reshape(1024, 1032)

    h = acc + b1[...]
    mu = jnp.mean(h, axis=1, keepdims=True)
    d = h - mu
    var = jnp.mean(d * d, axis=1, keepdims=True)
    h = d / jnp.sqrt(var + _EPS) * lng[...] + lnb[...]
    h = h * jax.nn.sigmoid(h)
    logits = jnp.dot(h, w2[...], preferred_element_type=jnp.float32) + b2[...]
    pred_ref[0] = jnp.argmax(logits, axis=1).astype(jnp.int32).reshape(8, 128)
    norm_ref[0] = jnp.sqrt(jnp.sum(logits * logits, axis=1)).reshape(8, 128)


def _mlp_stats(feats, w1s, b1, lng, lnb, w2, b2):
    B = feats[0].shape[0]
    in_specs = []
    for s in range(8):
        res = _RES[s]
        f = 128 // res
        cr = max(1, 8 // f)
        C = feats[s].shape[-1]
        if f >= 8:
            idx = functools.partial(lambda f_, b, t: (b, (t * 8) // f_, 0, 0), f)
        else:
            idx = lambda b, t: (b, t, 0, 0)
        in_specs.append(pl.BlockSpec((1, cr, res, C), idx))
    for s in range(8):
        C = feats[s].shape[-1]
        in_specs.append(pl.BlockSpec((C, 1032), lambda b, t: (0, 0)))
    for n in (1032, 1032, 1032, None, _NC):
        if n is None:
            in_specs.append(pl.BlockSpec((1032, _NC), lambda b, t: (0, 0)))
        else:
            in_specs.append(pl.BlockSpec((1, n), lambda b, t: (0, 0)))
    out_specs = [
        pl.BlockSpec((1, 8, 128), lambda b, t: (b, t, 0)),
        pl.BlockSpec((1, 8, 128), lambda b, t: (b, t, 0)),
    ]
    return pl.pallas_call(
        _mlp_stats_kernel,
        grid=(B, 16),
        in_specs=in_specs,
        out_specs=out_specs,
        out_shape=[
            jax.ShapeDtypeStruct((B, 128, 128), jnp.int32),
            jax.ShapeDtypeStruct((B, 128, 128), jnp.float32),
        ],
        compiler_params=pltpu.CompilerParams(
            dimension_semantics=("parallel", "arbitrary")),
    )(*feats, *w1s, b1[None], lng[None], lnb[None], w2, b2[None])


def _select_kernel(pred_ref, norm_ref, idx_ref, counts_ref):
    pred = pred_ref[0]
    norms = norm_ref[0]

    def hist_body(ct, _):
        cls = ct * 128 + jax.lax.broadcasted_iota(jnp.int32, (1, 1, 128), 2)
        m = (pred[:, :, None] == cls).astype(jnp.int32)
        counts_ref[pl.ds(ct, 1), :] = m.sum(axis=0).sum(axis=0)[None, :]
        return 0

    jax.lax.fori_loop(0, 8, hist_body, 0)
    counts = counts_ref[...]
    cg = (jax.lax.broadcasted_iota(jnp.int32, (8, 128), 0) * 128 +
          jax.lax.broadcasted_iota(jnp.int32, (8, 128), 1))
    maxc = jnp.max(counts)
    mode = jnp.min(jnp.where(counts == maxc, cg, jnp.int32(2**30)))

    mask = pred == mode
    cnt = jnp.sum(mask.astype(jnp.int32))
    k = (cnt - 1) // 2
    nbits = pltpu.bitcast(norms, jnp.int32)

    def bit_body(i, p):
        t = p | (jnp.int32(1) << (30 - i))
        c = jnp.sum((mask & (nbits < t)).astype(jnp.int32))
        return jnp.where(c <= k, t, p)

    med_bits = jax.lax.fori_loop(0, 31, bit_body, jnp.int32(0))
    med = pltpu.bitcast(jnp.full((8, 128), med_bits, jnp.int32), jnp.float32)[0, 0]

    diff = jnp.where(mask, jnp.abs(norms - med), jnp.float32(np.inf))
    tok = (jax.lax.broadcasted_iota(jnp.int32, (128, 128), 0) * 128 +
           jax.lax.broadcasted_iota(jnp.int32, (128, 128), 1))
    mind = jnp.min(diff)
    sel = jnp.min(jnp.where(diff == mind, tok, jnp.int32(2**30)))
    idx_ref[0] = jnp.full((1, 1), sel, jnp.int32)


def _select(pred, norms):
    B = pred.shape[0]
    return pl.pallas_call(
        _select_kernel,
        grid=(B,),
        in_specs=[
            pl.BlockSpec((1, 128, 128), lambda b: (b, 0, 0)),
            pl.BlockSpec((1, 128, 128), lambda b: (b, 0, 0)),
        ],
        out_specs=pl.BlockSpec((1, 1, 1), lambda b: (b, 0, 0)),
        out_shape=jax.ShapeDtypeStruct((B, 1, 1), jnp.int32),
        scratch_shapes=[pltpu.VMEM((8, 128), jnp.int32)],
        compiler_params=pltpu.CompilerParams(
            dimension_semantics=("arbitrary",)),
    )(pred, norms)


def _final_kernel(fv_ref, w1_ref, b1_ref, lng_ref, lnb_ref, w2_ref, b2_ref,
                  o_ref):
    h = jnp.dot(fv_ref[...], w1_ref[...],
                preferred_element_type=jnp.float32) + b1_ref[...]
    mu = jnp.mean(h, axis=1, keepdims=True)
    d = h - mu
    var = jnp.mean(d * d, axis=1, keepdims=True)
    h = d / jnp.sqrt(var + _EPS) * lng_ref[...] + lnb_ref[...]
    h = h * jax.nn.sigmoid(h)
    o_ref[...] = jnp.dot(h, w2_ref[...],
                         preferred_element_type=jnp.float32) + b2_ref[...]


def _final(fv, w1, b1, lng, lnb, w2, b2):
    B = fv.shape[0]
    return pl.pallas_call(
        _final_kernel,
        out_shape=jax.ShapeDtypeStruct((B, _NC), jnp.float32),
    )(fv, w1, b1[None], lng[None], lnb[None], w2, b2[None])


def kernel(x, params):
    B = x.shape[0]
    gray = (x * jnp.asarray(_GRAY)[None, :, None, None]).sum(1)

    feats = []
    for s, (blk, (oc, ks, st, pd)) in enumerate(zip(params['blocks'],
                                                    _CONV_SPECS)):
        res = _RES[s]
        if s < 7:
            p = gray.reshape(B, res, ks, res, ks).transpose(0, 1, 3, 2, 4)
            p = p.reshape(B * res * res, ks * ks)
            wt = blk['w'].reshape(oc, ks * ks)
        else:
            gp = jnp.pad(gray, ((0, 0), (1, 1), (1, 1)))
            cols = [gp[:, di:di + 256:2, dj:dj + 256:2]
                    for di in range(3) for dj in range(3)]
            p = jnp.stack(cols, axis=-1).reshape(B * res * res, 9)
            p = jnp.pad(p, ((0, 0), (0, 7)))
            wt = jnp.pad(blk['w'].reshape(oc, 9), ((0, 0), (0, 7)))
        scale = blk['gamma'] * jax.lax.rsqrt(blk['var'] + _EPS)
        shift = blk['beta'] - blk['mean'] * scale
        mt, nt, kt = _CONV_TILES[s]
        kf = _mm_bn_silu(p, wt, scale, shift, mt, nt, kt)
        kf = kf.reshape(B, res, res, oc)
        f = 256 // res
        rc = x.reshape(B, 3, res, f, res, f).mean(axis=(3, 5))
        rc = rc.transpose(0, 2, 3, 1)
        feats.append(jnp.concatenate([rc, kf], axis=-1))

    m = params['mlp']
    w1s, off = [], 0
    for s in range(8):
        C = feats[s].shape[-1]
        w1s.append(m['w1'][off:off + C])
        off += C

    pred, norms = _mlp_stats(feats, w1s, m['b1'], m['ln_g'], m['ln_b'],
                             m['w2'], m['b2'])

    idx = _select(pred, norms)[:, 0, 0]

    r, c = idx // 128, idx % 128
    pieces = []
    for s in range(8):
        res = _RES[s]
        f = 128 // res
        flat = feats[s].reshape(B, res * res, -1)
        cell = (r // f) * res + (c // f)
        pieces.append(jnp.take_along_axis(flat, cell[:, None, None], axis=1)[:, 0])
    fv = jnp.concatenate(pieces, axis=-1)

    return _final(fv, m['w1'], m['b1'], m['ln_g'], m['ln_b'], m['w2'], m['b2'])


# speedup vs baseline: 1.0721x; 1.0721x over previous
import functools

import jax
import jax.numpy as jnp
import numpy as np
from jax.experimental import pallas as pl
from jax.experimental.pallas import tpu as pltpu

_CONV_SPECS = [(1024, 256, 256, 0), (512, 128, 128, 0), (256, 64, 64, 0),
               (128, 32, 32, 0), (64, 16, 16, 0), (32, 8, 8, 0),
               (16, 4, 4, 0), (8, 3, 2, 1)]
_NC = 1000
_EPS = 1e-5
_GRAY = np.array([0.299, 0.587, 0.114], np.float32)
_RES = [1, 2, 4, 8, 16, 32, 64, 128]


def _mm_bn_silu_kernel(nk, p_ref, w_ref, s_ref, t_ref, o_ref, acc_ref):
    k = pl.program_id(2)

    @pl.when(k == 0)
    def _():
        acc_ref[...] = jnp.zeros_like(acc_ref)

    if w_ref.ndim == 4:
        ks = w_ref.shape[3]
        for j in range(w_ref.shape[2]):
            acc_ref[...] += jax.lax.dot_general(
                p_ref[:, j * ks:(j + 1) * ks], w_ref[:, 0, j, :],
                (((1,), (1,)), ((), ())), preferred_element_type=jnp.float32)
    else:
        acc_ref[...] += jnp.dot(p_ref[...], w_ref[...],
                                preferred_element_type=jnp.float32)

    @pl.when(k == nk - 1)
    def _():
        y = acc_ref[...] * s_ref[...] + t_ref[...]
        o_ref[...] = y * jax.nn.sigmoid(y)


def _mm_bn_silu(patches, wt, scale, shift, mt, nt, kt):
    M, K = patches.shape
    if wt.ndim == 4:
        N, ks = wt.shape[0], wt.shape[3]
        assert kt == 8 * ks and K == wt.shape[2] * ks
        w_spec = pl.BlockSpec((nt, 1, 8, ks), lambda m, n, k: (n, 0, k, 0))
    else:
        N = wt.shape[1]
        w_spec = pl.BlockSpec((kt, nt), lambda m, n, k: (k, n))
    nm, nn, nk = M // mt, N // nt, K // kt
    return pl.pallas_call(
        functools.partial(_mm_bn_silu_kernel, nk),
        grid=(nm, nn, nk),
        in_specs=[
            pl.BlockSpec((mt, kt), lambda m, n, k: (m, k)),
            w_spec,
            pl.BlockSpec((1, nt), lambda m, n, k: (0, n)),
            pl.BlockSpec((1, nt), lambda m, n, k: (0, n)),
        ],
        out_specs=pl.BlockSpec((mt, nt), lambda m, n, k: (m, n)),
        out_shape=jax.ShapeDtypeStruct((M, N), jnp.float32),
        scratch_shapes=[pltpu.VMEM((mt, nt), jnp.float32)],
        compiler_params=pltpu.CompilerParams(
            dimension_semantics=("parallel", "parallel", "arbitrary")),
    )(patches, wt, scale[None], shift[None])


_CONV_TILES = [(2, 512, 2048), (8, 256, 1024), (16, 256, 4096),
               (64, 128, 1024), (256, 64, 256), (1024, 32, 64),
               (4096, 16, 16), (16384, 8, 16)]


def _mlp_stats_kernel(*refs):
    feats = refs[0:8]
    w1s = refs[8:16]
    b1, lng, lnb, w2, b2 = refs[16:21]
    pred_ref, norm_ref = refs[21:23]

    acc = jnp.zeros((1024, 1032), jnp.float32)
    for s in range(8):
        res = _RES[s]
        f = 128 // res
        cr = max(1, 8 // f)
        rr = min(8, f)
        cells = feats[s][0].reshape(cr * res, -1)
        g = jnp.dot(cells, w1s[s][...], preferred_element_type=jnp.float32)
        if f == 1:
            acc += g
        else:
            g = g.reshape(cr, res, 1032)
            g = jnp.broadcast_to(g[:, None, :, None, :], (cr, rr, res, f, 1032))
            acc += g.reshape(1024, 1032)

    h = acc + b1[...]
    mu = jnp.mean(h, axis=1, keepdims=True)
    d = h - mu
    var = jnp.mean(d * d, axis=1, keepdims=True)
    h = d / jnp.sqrt(var + _EPS) * lng[...] + lnb[...]
    h = h * jax.nn.sigmoid(h)
    logits = jnp.dot(h, w2[...], preferred_element_type=jnp.float32) + b2[...]
    pred_ref[0] = jnp.argmax(logits, axis=1).astype(jnp.int32).reshape(8, 128)
    norm_ref[0] = jnp.sqrt(jnp.sum(logits * logits, axis=1)).reshape(8, 128)


def _mlp_stats(feats, w1s, b1, lng, lnb, w2, b2):
    B = feats[0].shape[0]
    in_specs = []
    for s in range(8):
        res = _RES[s]
        f = 128 // res
        cr = max(1, 8 // f)
        C = feats[s].shape[-1]
        if f >= 8:
            idx = functools.partial(lambda f_, b, t: (b, (t * 8) // f_, 0, 0), f)
        else:
            idx = lambda b, t: (b, t, 0, 0)
        in_specs.append(pl.BlockSpec((1, cr, res, C), idx))
    for s in range(8):
        C = feats[s].shape[-1]
        in_specs.append(pl.BlockSpec((C, 1032), lambda b, t: (0, 0)))
    for n in (1032, 1032, 1032, None, _NC):
        if n is None:
            in_specs.append(pl.BlockSpec((1032, _NC), lambda b, t: (0, 0)))
        else:
            in_specs.append(pl.BlockSpec((1, n), lambda b, t: (0, 0)))
    out_specs = [
        pl.BlockSpec((1, 8, 128), lambda b, t: (b, t, 0)),
        pl.BlockSpec((1, 8, 128), lambda b, t: (b, t, 0)),
    ]
    return pl.pallas_call(
        _mlp_stats_kernel,
        grid=(B, 16),
        in_specs=in_specs,
        out_specs=out_specs,
        out_shape=[
            jax.ShapeDtypeStruct((B, 128, 128), jnp.int32),
            jax.ShapeDtypeStruct((B, 128, 128), jnp.float32),
        ],
        compiler_params=pltpu.CompilerParams(
            dimension_semantics=("parallel", "arbitrary")),
    )(*feats, *w1s, b1[None], lng[None], lnb[None], w2, b2[None])


def _select_kernel(pred_ref, norm_ref, idx_ref, counts_ref):
    pred = pred_ref[0]
    norms = norm_ref[0]

    def hist_body(ct, _):
        cls = ct * 128 + jax.lax.broadcasted_iota(jnp.int32, (1, 1, 128), 2)
        m = (pred[:, :, None] == cls).astype(jnp.int32)
        counts_ref[pl.ds(ct, 1), :] = m.sum(axis=0).sum(axis=0)[None, :]
        return 0

    jax.lax.fori_loop(0, 8, hist_body, 0)
    counts = counts_ref[...]
    cg = (jax.lax.broadcasted_iota(jnp.int32, (8, 128), 0) * 128 +
          jax.lax.broadcasted_iota(jnp.int32, (8, 128), 1))
    maxc = jnp.max(counts)
    mode = jnp.min(jnp.where(counts == maxc, cg, jnp.int32(2**30)))

    mask = pred == mode
    cnt = jnp.sum(mask.astype(jnp.int32))
    k = (cnt - 1) // 2
    nbits = pltpu.bitcast(norms, jnp.int32)

    def bit_body(i, p):
        t = p | (jnp.int32(1) << (30 - i))
        c = jnp.sum((mask & (nbits < t)).astype(jnp.int32))
        return jnp.where(c <= k, t, p)

    med_bits = jax.lax.fori_loop(0, 31, bit_body, jnp.int32(0))
    med = pltpu.bitcast(jnp.full((8, 128), med_bits, jnp.int32), jnp.float32)[0, 0]

    diff = jnp.where(mask, jnp.abs(norms - med), jnp.float32(np.inf))
    tok = (jax.lax.broadcasted_iota(jnp.int32, (128, 128), 0) * 128 +
           jax.lax.broadcasted_iota(jnp.int32, (128, 128), 1))
    mind = jnp.min(diff)
    sel = jnp.min(jnp.where(diff == mind, tok, jnp.int32(2**30)))
    idx_ref[0] = jnp.full((1, 1), sel, jnp.int32)


def _select(pred, norms):
    B = pred.shape[0]
    return pl.pallas_call(
        _select_kernel,
        grid=(B,),
        in_specs=[
            pl.BlockSpec((1, 128, 128), lambda b: (b, 0, 0)),
            pl.BlockSpec((1, 128, 128), lambda b: (b, 0, 0)),
        ],
        out_specs=pl.BlockSpec((1, 1, 1), lambda b: (b, 0, 0)),
        out_shape=jax.ShapeDtypeStruct((B, 1, 1), jnp.int32),
        scratch_shapes=[pltpu.VMEM((8, 128), jnp.int32)],
        compiler_params=pltpu.CompilerParams(
            dimension_semantics=("arbitrary",)),
    )(pred, norms)


def _final_kernel(fv_ref, w1_ref, b1_ref, lng_ref, lnb_ref, w2_ref, b2_ref,
                  o_ref):
    h = jnp.dot(fv_ref[...], w1_ref[...],
                preferred_element_type=jnp.float32) + b1_ref[...]
    mu = jnp.mean(h, axis=1, keepdims=True)
    d = h - mu
    var = jnp.mean(d * d, axis=1, keepdims=True)
    h = d / jnp.sqrt(var + _EPS) * lng_ref[...] + lnb_ref[...]
    h = h * jax.nn.sigmoid(h)
    o_ref[...] = jnp.dot(h, w2_ref[...],
                         preferred_element_type=jnp.float32) + b2_ref[...]


def _final(fv, w1, b1, lng, lnb, w2, b2):
    B = fv.shape[0]
    return pl.pallas_call(
        _final_kernel,
        out_shape=jax.ShapeDtypeStruct((B, _NC), jnp.float32),
    )(fv, w1, b1[None], lng[None], lnb[None], w2, b2[None])


def kernel(x, params):
    B = x.shape[0]
    gray = (x * jnp.asarray(_GRAY)[None, :, None, None]).sum(1)

    feats = []
    for s, (blk, (oc, ks, st, pd)) in enumerate(zip(params['blocks'],
                                                    _CONV_SPECS)):
        res = _RES[s]
        if s < 7:
            p = gray.reshape(B, res, ks, res, ks).transpose(0, 1, 3, 2, 4)
            p = p.reshape(B * res * res, ks * ks)
            wt = blk['w'] if s < 2 else blk['w'].reshape(oc, ks * ks).T
        else:
            gp = jnp.pad(gray, ((0, 0), (1, 1), (1, 1)))
            cols = [gp[:, di:di + 256:2, dj:dj + 256:2]
                    for di in range(3) for dj in range(3)]
            p = jnp.stack(cols, axis=-1).reshape(B * res * res, 9)
            p = jnp.pad(p, ((0, 0), (0, 7)))
            wt = jnp.pad(blk['w'].reshape(oc, 9).T, ((0, 7), (0, 0)))
        scale = blk['gamma'] * jax.lax.rsqrt(blk['var'] + _EPS)
        shift = blk['beta'] - blk['mean'] * scale
        mt, nt, kt = _CONV_TILES[s]
        kf = _mm_bn_silu(p, wt, scale, shift, mt, nt, kt)
        kf = kf.reshape(B, res, res, oc)
        f = 256 // res
        rc = x.reshape(B, 3, res, f, res, f).mean(axis=(3, 5))
        rc = rc.transpose(0, 2, 3, 1)
        feats.append(jnp.concatenate([rc, kf], axis=-1))

    m = params['mlp']
    w1s, off = [], 0
    for s in range(8):
        C = feats[s].shape[-1]
        w1s.append(m['w1'][off:off + C])
        off += C

    pred, norms = _mlp_stats(feats, w1s, m['b1'], m['ln_g'], m['ln_b'],
                             m['w2'], m['b2'])

    idx = _select(pred, norms)[:, 0, 0]

    r, c = idx // 128, idx % 128
    pieces = []
    for s in range(8):
        res = _RES[s]
        f = 128 // res
        flat = feats[s].reshape(B, res * res, -1)
        cell = (r // f) * res + (c // f)
        pieces.append(jnp.take_along_axis(flat, cell[:, None, None], axis=1)[:, 0])
    fv = jnp.concatenate(pieces, axis=-1)

    return _final(fv, m['w1'], m['b1'], m['ln_g'], m['ln_b'], m['w2'], m['b2'])
